# baseline (speedup 1.0000x reference)
import functools

import jax
import jax.numpy as jnp
from jax import lax
from jax.experimental import pallas as pl
from jax.experimental.pallas import tpu as pltpu
from jax.experimental.pallas import tpu_sc as plsc

N_NODES = 10000
NP = 10112
N_EDGES = 320000
NW = 32
EPW = 10240
EPAD = NW * EPW
CHUNK_BATCHES = 40
CHUNK_EDGES = CHUNK_BATCHES * 128
N_CHUNKS = EPAD // CHUNK_EDGES
MAXQ = 4
PAIRS = CHUNK_BATCHES // 2
MB = 1000

_mesh = plsc.VectorSubcoreMesh(core_axis_name="c", subcore_axis_name="s")


def _make_agg_kernel(D):
    @functools.partial(
        pl.kernel,
        out_type=jax.ShapeDtypeStruct((2, NP, D), jnp.float32),
        mesh=_mesh,
        scratch_types=[
            pltpu.VMEM((CHUNK_BATCHES, 128), jnp.int32),
            pltpu.VMEM((CHUNK_BATCHES, 128), jnp.int32),
            pltpu.VMEM((128, D), jnp.float32),
            pltpu.VMEM((128, D), jnp.float32),
            pltpu.VMEM_SHARED((NP, D), jnp.float32),
            pltpu.SemaphoreType.DMA,
            pltpu.SemaphoreType.DMA,
        ],
    )
    def agg_kernel(y_hbm, src_hbm, dst_hbm, out_hbm,
                   srci, dsti, rows0, rows1, acc, semg, sems):
        c = lax.axis_index("c")
        s = lax.axis_index("s")
        wid = s * 2 + c

        with jax.named_scope("agg_zero"):
            z16 = jnp.zeros((16,), jnp.float32)

            def zrow(r, carry):
                for j in range(D // 16):
                    rows0[r, pl.ds(j * 16, 16)] = z16
                return carry

            lax.fori_loop(0, 128, zrow, 0)
            base = s * 632
            for k in range(4):
                pltpu.sync_copy(rows0, acc.at[pl.ds(base + k * 128, 128)])
            pltpu.sync_copy(rows0.at[pl.ds(0, 120)],
                            acc.at[pl.ds(base + 512, 120)])
            plsc.subcore_barrier()

        def wait_g(buf):
            pltpu.make_async_copy(y_hbm.at[srci.at[0]], buf, semg).wait()

        def drain_s(buf):
            pltpu.make_async_copy(buf, acc.at[dsti.at[0]], sems).wait()

        edge_scope = jax.named_scope("agg_edges")
        edge_scope.__enter__()
        for q in range(MAXQ):
            hrow = MAXQ * s + q

            @pl.when(c == 0)
            def _chunk():
                pltpu.sync_copy(src_hbm.at[hrow], srci)
                pltpu.sync_copy(dst_hbm.at[hrow], dsti)
                pltpu.async_copy(y_hbm.at[srci.at[0]], rows0, semg)

                def body(k, carry):
                    b0 = 2 * k
                    b1 = 2 * k + 1
                    wait_g(rows0)

                    @pl.when(k > 0)
                    def _():
                        drain_s(rows1)

                    pltpu.async_copy(rows0, acc.at[dsti.at[b0]], sems,
                                     add=True)
                    pltpu.async_copy(y_hbm.at[srci.at[b1]], rows1, semg)
                    wait_g(rows1)
                    drain_s(rows0)
                    pltpu.async_copy(rows1, acc.at[dsti.at[b1]], sems,
                                     add=True)

                    @pl.when(k < PAIRS - 1)
                    def _():
                        pltpu.async_copy(y_hbm.at[srci.at[b0 + 2]], rows0,
                                         semg)

                    return carry

                lax.fori_loop(0, PAIRS, body, 0)
                drain_s(rows1)
        plsc.subcore_barrier()
        edge_scope.__exit__(None, None, None)
        with jax.named_scope("agg_readout"):
            @pl.when(c == 0)
            def _():
                pltpu.sync_copy(acc.at[pl.ds(s * 632, 632)],
                                out_hbm.at[0, pl.ds(s * 632, 632)])

    return agg_kernel


_agg128 = _make_agg_kernel(128)


@functools.partial(
    pl.kernel,
    out_type=jax.ShapeDtypeStruct((NP, 128), jnp.float32),
    mesh=_mesh,
    scratch_types=[
        pltpu.VMEM((CHUNK_BATCHES, 128), jnp.int32),
        pltpu.VMEM((128, 128), jnp.float32),
        pltpu.VMEM_SHARED((NP, 128), jnp.float32),
        pltpu.SemaphoreType.DMA,
    ],
)
def _deg_kernel(dst_hbm, out_hbm, dsti, ones, acc, sems):
    c = lax.axis_index("c")
    s = lax.axis_index("s")
    o16 = jnp.ones((16,), jnp.float32)
    z16 = jnp.zeros((16,), jnp.float32)
    base = s * 632

    def fill(v):
        def row(r, carry):
            for j in range(8):
                ones[r, pl.ds(j * 16, 16)] = v
            return carry
        return row

    lax.fori_loop(0, 128, fill(z16), 0)
    for k in range(4):
        pltpu.sync_copy(ones, acc.at[pl.ds(base + k * 128, 128)])
    pltpu.sync_copy(ones.at[pl.ds(0, 120)], acc.at[pl.ds(base + 512, 120)])
    lax.fori_loop(0, 128, fill(o16), 0)
    plsc.subcore_barrier()

    @pl.when(c == 0)
    def _():
        for q in range(MAXQ):
            hrow = MAXQ * s + q
            pltpu.sync_copy(dst_hbm.at[hrow], dsti)

            def body(b, carry):
                pltpu.async_copy(ones, acc.at[dsti.at[b]], sems, add=True)
                return carry

            lax.fori_loop(0, CHUNK_BATCHES, body, 0)

            def drain(b, carry):
                pltpu.make_async_copy(ones, acc.at[dsti.at[0]], sems).wait()
                return carry

            lax.fori_loop(0, CHUNK_BATCHES, drain, 0)
    plsc.subcore_barrier()

    @pl.when(c == 0)
    def _():
        pltpu.sync_copy(acc.at[pl.ds(s * 632, 632)],
                        out_hbm.at[pl.ds(s * 632, 632)])


def _dinv_y0_body(p_ref, x_ref, w_ref, y_ref, dv_ref):
    deg = p_ref[:, 0:1] + 1.0
    dvb = jnp.broadcast_to(lax.rsqrt(deg), (MB, 128))
    h = jnp.dot(x_ref[...], w_ref[...], preferred_element_type=jnp.float32)
    y_ref[...] = h * dvb
    dv_ref[...] = dvb


def _mk_mid_body(H):
    def body(p_ref, y_ref, dv_ref, m_ref, b_ref, w_ref, o_ref):
        agg = p_ref[0] + y_ref[...]
        pre = agg * dv_ref[...] + b_ref[...]
        t = jnp.maximum(pre, 0.0) * m_ref[...]
        h = jnp.dot(t, w_ref[...], preferred_element_type=jnp.float32)
        o_ref[...] = h * dv_ref[:, 0:H]
    return body


def _final_body(p_ref, y_ref, dv_ref, b_ref, o_ref):
    agg = p_ref[0, :, 0:64] + y_ref[:, 0:64]
    o_ref[...] = agg * dv_ref[:, 0:64] + b_ref[...]


def _row_block(width):
    return pl.BlockSpec((MB, width), lambda i: (i, 0))


def _p_block(width):
    return pl.BlockSpec((1, MB, width), lambda i: (0, i, 0))


def _full_block(r, c):
    return pl.BlockSpec((r, c), lambda i: (0, 0))


def kernel(x, edge_index, W0, b0, W1, b1, W2, b2):
    f32 = jnp.float32
    pad = EPAD - N_EDGES
    src = jnp.concatenate([edge_index[0], jnp.zeros((pad,), jnp.int32)])
    dst = jnp.concatenate([edge_index[1],
                           jnp.full((pad,), N_NODES, jnp.int32)])
    srcm = src.reshape(N_CHUNKS, CHUNK_BATCHES, 128)
    dstm = dst.reshape(N_CHUNKS, CHUNK_BATCHES, 128)
    W2p = jnp.pad(W2, ((0, 0), (0, 64)))

    dkey = jax.random.key(42)
    mask0 = jax.random.bernoulli(
        jax.random.fold_in(dkey, 0), 0.5, (N_NODES, 128)).astype(f32) * 2.0
    mask1 = jax.random.bernoulli(
        jax.random.fold_in(dkey, 1), 0.5, (N_NODES, 128)).astype(f32) * 2.0

    b0r = b0.reshape(1, 128)
    b1r = b1.reshape(1, 128)
    b2r = b2.reshape(1, 64)

    deg_parts = _deg_kernel(dstm)

    grid = N_NODES // MB
    y0, dinvb = pl.pallas_call(
        _dinv_y0_body,
        grid=(grid,),
        in_specs=[_row_block(128), _row_block(128), _full_block(128, 128)],
        out_specs=[_row_block(128), _row_block(128)],
        out_shape=[jax.ShapeDtypeStruct((N_NODES, 128), f32),
                   jax.ShapeDtypeStruct((N_NODES, 128), f32)],
    )(deg_parts, x, W0)

    p0 = _agg128(y0, srcm, dstm)
    y1 = pl.pallas_call(
        _mk_mid_body(128),
        grid=(grid,),
        in_specs=[_p_block(128), _row_block(128), _row_block(128),
                  _row_block(128), _full_block(1, 128), _full_block(128, 128)],
        out_specs=_row_block(128),
        out_shape=jax.ShapeDtypeStruct((N_NODES, 128), f32),
    )(p0, y0, dinvb, mask0, b0r, W1)

    p1 = _agg128(y1, srcm, dstm)
    y2 = pl.pallas_call(
        _mk_mid_body(128),
        grid=(grid,),
        in_specs=[_p_block(128), _row_block(128), _row_block(128),
                  _row_block(128), _full_block(1, 128), _full_block(128, 128)],
        out_specs=_row_block(128),
        out_shape=jax.ShapeDtypeStruct((N_NODES, 128), f32),
    )(p1, y1, dinvb, mask1, b1r, W2p)

    p2 = _agg128(y2, srcm, dstm)
    out = pl.pallas_call(
        _final_body,
        grid=(grid,),
        in_specs=[_p_block(128), _row_block(128), _row_block(128),
                  _full_block(1, 64)],
        out_specs=_row_block(64),
        out_shape=jax.ShapeDtypeStruct((N_NODES, 64), f32),
    )(p2, y2, dinvb, b2r)
    return out


# speedup vs baseline: 1.4002x; 1.4002x over previous
import functools

import jax
import jax.numpy as jnp
from jax import lax
from jax.experimental import pallas as pl
from jax.experimental.pallas import tpu as pltpu
from jax.experimental.pallas import tpu_sc as plsc

N_NODES = 10000
NP = 10112
N_EDGES = 320000
NW = 32
EPW = 10240
EPAD = NW * EPW
CHUNK_BATCHES = 40
UNIT_BATCHES = 20
UNIT_EDGES = UNIT_BATCHES * 128
N_UNITS = EPAD // UNIT_EDGES
UNITS_SC0 = 7
MAXQ = UNITS_SC0
PAIRS = UNIT_BATCHES // 2
DEG_MAXQ = 4
MB = 1000

_mesh = plsc.VectorSubcoreMesh(core_axis_name="c", subcore_axis_name="s")


def _make_agg_kernel(D):
    @functools.partial(
        pl.kernel,
        out_type=jax.ShapeDtypeStruct((2, NP, D), jnp.float32),
        mesh=_mesh,
        scratch_types=[
            pltpu.VMEM((UNIT_BATCHES, 128), jnp.int32),
            pltpu.VMEM((UNIT_BATCHES, 128), jnp.int32),
            pltpu.VMEM((128, D), jnp.float32),
            pltpu.VMEM((128, D), jnp.float32),
            pltpu.VMEM_SHARED((NP, D), jnp.float32),
            pltpu.SemaphoreType.DMA,
            pltpu.SemaphoreType.DMA,
        ],
    )
    def agg_kernel(y_hbm, src_hbm, dst_hbm, out_hbm,
                   srci, dsti, rows0, rows1, acc, semg, sems):
        c = lax.axis_index("c")
        s = lax.axis_index("s")
        wid = s * 2 + c

        with jax.named_scope("agg_zero"):
            z16 = jnp.zeros((16,), jnp.float32)

            def zrow(r, carry):
                for j in range(D // 16):
                    rows0[r, pl.ds(j * 16, 16)] = z16
                return carry

            lax.fori_loop(0, 128, zrow, 0)
            base = s * 632
            for k in range(4):
                pltpu.sync_copy(rows0, acc.at[pl.ds(base + k * 128, 128)])
            pltpu.sync_copy(rows0.at[pl.ds(0, 120)],
                            acc.at[pl.ds(base + 512, 120)])
            plsc.subcore_barrier()

        def wait_g(buf):
            pltpu.make_async_copy(y_hbm.at[srci.at[0]], buf, semg).wait()

        def drain_s(buf):
            pltpu.make_async_copy(buf, acc.at[dsti.at[0]], sems).wait()

        edge_scope = jax.named_scope("agg_edges")
        edge_scope.__enter__()
        for q in range(MAXQ):
            hrow = jnp.where(c == 0, UNITS_SC0 * s + q, 16 * UNITS_SC0 + s)

            @pl.when((c == 0) | (q == 0))
            def _chunk():
                pltpu.sync_copy(src_hbm.at[hrow], srci)
                pltpu.sync_copy(dst_hbm.at[hrow], dsti)
                pltpu.async_copy(y_hbm.at[srci.at[0]], rows0, semg)

                def body(k, carry):
                    b0 = 2 * k
                    b1 = 2 * k + 1
                    wait_g(rows0)

                    @pl.when(k > 0)
                    def _():
                        drain_s(rows1)

                    pltpu.async_copy(rows0, acc.at[dsti.at[b0]], sems,
                                     add=True)
                    pltpu.async_copy(y_hbm.at[srci.at[b1]], rows1, semg)
                    wait_g(rows1)
                    drain_s(rows0)
                    pltpu.async_copy(rows1, acc.at[dsti.at[b1]], sems,
                                     add=True)

                    @pl.when(k < PAIRS - 1)
                    def _():
                        pltpu.async_copy(y_hbm.at[srci.at[b0 + 2]], rows0,
                                         semg)

                    return carry

                lax.fori_loop(0, PAIRS, body, 0)
                drain_s(rows1)
        plsc.subcore_barrier()
        edge_scope.__exit__(None, None, None)
        with jax.named_scope("agg_readout"):
            pltpu.sync_copy(acc.at[pl.ds(s * 632, 632)],
                            out_hbm.at[c, pl.ds(s * 632, 632)])

    return agg_kernel


_agg128 = _make_agg_kernel(128)


@functools.partial(
    pl.kernel,
    out_type=jax.ShapeDtypeStruct((NP, 128), jnp.float32),
    mesh=_mesh,
    scratch_types=[
        pltpu.VMEM((CHUNK_BATCHES, 128), jnp.int32),
        pltpu.VMEM((128, 128), jnp.float32),
        pltpu.VMEM_SHARED((NP, 128), jnp.float32),
        pltpu.SemaphoreType.DMA,
    ],
)
def _deg_kernel(dst_hbm, out_hbm, dsti, ones, acc, sems):
    c = lax.axis_index("c")
    s = lax.axis_index("s")
    o16 = jnp.ones((16,), jnp.float32)
    z16 = jnp.zeros((16,), jnp.float32)
    base = s * 632

    def fill(v):
        def row(r, carry):
            for j in range(8):
                ones[r, pl.ds(j * 16, 16)] = v
            return carry
        return row

    lax.fori_loop(0, 128, fill(z16), 0)
    for k in range(4):
        pltpu.sync_copy(ones, acc.at[pl.ds(base + k * 128, 128)])
    pltpu.sync_copy(ones.at[pl.ds(0, 120)], acc.at[pl.ds(base + 512, 120)])
    lax.fori_loop(0, 128, fill(o16), 0)
    plsc.subcore_barrier()

    @pl.when(c == 0)
    def _():
        for q in range(DEG_MAXQ):
            hrow = DEG_MAXQ * s + q
            pltpu.sync_copy(dst_hbm.at[hrow], dsti)

            def body(b, carry):
                pltpu.async_copy(ones, acc.at[dsti.at[b]], sems, add=True)
                return carry

            lax.fori_loop(0, CHUNK_BATCHES, body, 0)

            def drain(b, carry):
                pltpu.make_async_copy(ones, acc.at[dsti.at[0]], sems).wait()
                return carry

            lax.fori_loop(0, CHUNK_BATCHES, drain, 0)
    plsc.subcore_barrier()

    @pl.when(c == 0)
    def _():
        pltpu.sync_copy(acc.at[pl.ds(s * 632, 632)],
                        out_hbm.at[pl.ds(s * 632, 632)])


def _dinv_y0_body(p_ref, x_ref, w_ref, y_ref, dv_ref):
    deg = p_ref[:, 0:1] + 1.0
    dvb = jnp.broadcast_to(lax.rsqrt(deg), (MB, 128))
    h = jnp.dot(x_ref[...], w_ref[...], preferred_element_type=jnp.float32)
    y_ref[...] = h * dvb
    dv_ref[...] = dvb


def _mk_mid_body(H):
    def body(p_ref, y_ref, dv_ref, m_ref, b_ref, w_ref, o_ref):
        agg = p_ref[0] + p_ref[1] + y_ref[...]
        pre = agg * dv_ref[...] + b_ref[...]
        t = jnp.maximum(pre, 0.0) * m_ref[...]
        h = jnp.dot(t, w_ref[...], preferred_element_type=jnp.float32)
        o_ref[...] = h * dv_ref[:, 0:H]
    return body


def _final_body(p_ref, y_ref, dv_ref, b_ref, o_ref):
    agg = p_ref[0, :, 0:64] + p_ref[1, :, 0:64] + y_ref[:, 0:64]
    o_ref[...] = agg * dv_ref[:, 0:64] + b_ref[...]


def _row_block(width):
    return pl.BlockSpec((MB, width), lambda i: (i, 0))


def _p_block(width):
    return pl.BlockSpec((2, MB, width), lambda i: (0, i, 0))


def _full_block(r, c):
    return pl.BlockSpec((r, c), lambda i: (0, 0))


def kernel(x, edge_index, W0, b0, W1, b1, W2, b2):
    f32 = jnp.float32
    pad = EPAD - N_EDGES
    src = jnp.concatenate([edge_index[0], jnp.zeros((pad,), jnp.int32)])
    dst = jnp.concatenate([edge_index[1],
                           jnp.full((pad,), N_NODES, jnp.int32)])
    srcm = src.reshape(N_UNITS, UNIT_BATCHES, 128)
    dstm = dst.reshape(N_UNITS, UNIT_BATCHES, 128)
    dstm40 = dst.reshape(EPAD // (CHUNK_BATCHES * 128), CHUNK_BATCHES, 128)
    W2p = jnp.pad(W2, ((0, 0), (0, 64)))

    dkey = jax.random.key(42)
    mask0 = jax.random.bernoulli(
        jax.random.fold_in(dkey, 0), 0.5, (N_NODES, 128)).astype(f32) * 2.0
    mask1 = jax.random.bernoulli(
        jax.random.fold_in(dkey, 1), 0.5, (N_NODES, 128)).astype(f32) * 2.0

    b0r = b0.reshape(1, 128)
    b1r = b1.reshape(1, 128)
    b2r = b2.reshape(1, 64)

    deg_parts = _deg_kernel(dstm40)

    grid = N_NODES // MB
    y0, dinvb = pl.pallas_call(
        _dinv_y0_body,
        grid=(grid,),
        in_specs=[_row_block(128), _row_block(128), _full_block(128, 128)],
        out_specs=[_row_block(128), _row_block(128)],
        out_shape=[jax.ShapeDtypeStruct((N_NODES, 128), f32),
                   jax.ShapeDtypeStruct((N_NODES, 128), f32)],
    )(deg_parts, x, W0)

    p0 = _agg128(y0, srcm, dstm)
    y1 = pl.pallas_call(
        _mk_mid_body(128),
        grid=(grid,),
        in_specs=[_p_block(128), _row_block(128), _row_block(128),
                  _row_block(128), _full_block(1, 128), _full_block(128, 128)],
        out_specs=_row_block(128),
        out_shape=jax.ShapeDtypeStruct((N_NODES, 128), f32),
    )(p0, y0, dinvb, mask0, b0r, W1)

    p1 = _agg128(y1, srcm, dstm)
    y2 = pl.pallas_call(
        _mk_mid_body(128),
        grid=(grid,),
        in_specs=[_p_block(128), _row_block(128), _row_block(128),
                  _row_block(128), _full_block(1, 128), _full_block(128, 128)],
        out_specs=_row_block(128),
        out_shape=jax.ShapeDtypeStruct((N_NODES, 128), f32),
    )(p1, y1, dinvb, mask1, b1r, W2p)

    p2 = _agg128(y2, srcm, dstm)
    out = pl.pallas_call(
        _final_body,
        grid=(grid,),
        in_specs=[_p_block(128), _row_block(128), _row_block(128),
                  _full_block(1, 64)],
        out_specs=_row_block(64),
        out_shape=jax.ShapeDtypeStruct((N_NODES, 64), f32),
    )(p2, y2, dinvb, b2r)
    return out
